# baseline (speedup 1.0000x reference)
import functools

import jax
import jax.numpy as jnp
from jax import lax
from jax.experimental import pallas as pl
from jax.experimental.pallas import tpu as pltpu
from jax.experimental.pallas import tpu_sc as plsc

B = 16384
D = 64
NF = 128
NU = 1000000
NC, NS = 2, 16
NW = NC * NS
BPW = B // NW

UBLK = 65536
UGRID = -(-NU // UBLK)
NMB = 16
MBLK = B // NMB


def _tc_body(w1_ref, tT_ref, mf_ref, wfT_ref, w2_ref, bf_ref, bfc_ref,
             ud_ref, qm_ref):
    ud_ref[...] = jnp.dot(w1_ref[...], tT_ref[...],
                          preferred_element_type=jnp.float32).reshape(UBLK)
    wm_row = jnp.dot(w2_ref[...], wfT_ref[...],
                     preferred_element_type=jnp.float32)
    qm = lax.dot_general(mf_ref[...], wm_row, (((1,), (1,)), ((), ())),
                         preferred_element_type=jnp.float32)
    c = jnp.sum(bf_ref[...] * w2_ref[...]) + bfc_ref[0, 0]
    qm_ref[...] = qm[:, 0] + c


_tc_dense = pl.pallas_call(
    _tc_body,
    grid=(UGRID,),
    in_specs=[
        pl.BlockSpec((1, D), lambda i: (0, 0)),
        pl.BlockSpec((D, UBLK), lambda i: (0, i)),
        pl.BlockSpec((MBLK, NF), lambda i: (jnp.minimum(i, NMB - 1), 0)),
        pl.BlockSpec((D, NF), lambda i: (0, 0)),
        pl.BlockSpec((1, D), lambda i: (0, 0)),
        pl.BlockSpec((1, D), lambda i: (0, 0)),
        pl.BlockSpec((1, 1), lambda i: (0, 0)),
    ],
    out_specs=[
        pl.BlockSpec((UBLK,), lambda i: (i,)),
        pl.BlockSpec((MBLK,), lambda i: (jnp.minimum(i, NMB - 1),)),
    ],
    out_shape=[
        jax.ShapeDtypeStruct((NU,), jnp.float32),
        jax.ShapeDtypeStruct((B,), jnp.float32),
    ],
    compiler_params=pltpu.CompilerParams(
        vmem_limit_bytes=100 * 1024 * 1024),
)


def _make_sc():
    mesh = plsc.VectorSubcoreMesh(core_axis_name="c", subcore_axis_name="s")

    @functools.partial(
        pl.kernel,
        mesh=mesh,
        compiler_params=pltpu.CompilerParams(use_tc_tiling_on_sc=False),
        out_type=jax.ShapeDtypeStruct((B,), jnp.float32),
        scratch_types=[
            pltpu.VMEM((BPW,), jnp.int32),
            pltpu.VMEM((BPW,), jnp.float32),
            pltpu.VMEM((BPW,), jnp.float32),
            pltpu.VMEM((BPW,), jnp.float32),
            pltpu.SemaphoreType.DMA,
        ],
    )
    def sc_k(udot_hbm, qm_hbm, idx_hbm, out_hbm,
             idx_v, ud_v, qm_v, out_v, sem_g):
        wid = lax.axis_index("s") * NC + lax.axis_index("c")
        base = wid * BPW

        pltpu.sync_copy(idx_hbm.at[pl.ds(base, BPW)], idx_v)
        gcp = pltpu.async_copy(udot_hbm.at[idx_v], ud_v, sem_g)
        pltpu.sync_copy(qm_hbm.at[pl.ds(base, BPW)], qm_v)
        gcp.wait()

        def body(g, carry):
            out_v[pl.ds(g * 16, 16)] = (ud_v[pl.ds(g * 16, 16)]
                                        + qm_v[pl.ds(g * 16, 16)])
            return carry

        lax.fori_loop(0, BPW // 16, body, 0)
        pltpu.sync_copy(out_v, out_hbm.at[pl.ds(base, BPW)])

    return sc_k


_sc_cache = []


def kernel(user_ids, movie_features, user_embedding, W_feat, b_feat, W_fc, b_fc):
    if not _sc_cache:
        _sc_cache.append(_make_sc())
    w1 = W_fc[:D, 0].reshape(1, D)
    w2 = W_fc[D:, 0].reshape(1, D)
    udot, qmovie = _tc_dense(w1, user_embedding.T, movie_features, W_feat.T,
                             w2, b_feat.reshape(1, D), b_fc.reshape(1, 1))
    return _sc_cache[0](udot, qmovie, user_ids.astype(jnp.int32))
